# baseline (speedup 1.0000x reference)
import functools

import jax
import jax.numpy as jnp
from jax import lax
from jax.experimental import pallas as pl
from jax.experimental.pallas import tpu as pltpu
from jax.experimental.pallas import tpu_sc as plsc

V = 1000000
F = 26
D = 32
B = 4096
BF = B * F

_NC = 2
_NS = 16
_NW = _NC * _NS

_TW = 512
_VFULL = (V // 128) * 128
_NST = _VFULL // _TW
_NPAIR = (_NST // _NW + 1 + 1) // 2
_TAIL = V - _VFULL
_OUTW = _TW * D // 2


def _shuffle(in_buf, out_buf, iota, iota_d):
    def body_d0(d0, _):
        dhalf = lax.rem(d0 + iota, D // 2)
        dvec = dhalf * 2
        for k in range(_TW // 16):
            jvec = iota + k * 16
            a = plsc.load_gather(in_buf, [dvec, jvec])
            b = plsc.load_gather(in_buf, [dvec + 1, jvec])
            p = plsc.pack(a, b, format=plsc.PackFormat.INTERLEAVED)
            w = plsc.bitcast(p, jnp.int32)
            plsc.store_scatter(out_buf, [iota_d + (k * 16 * D // 2) + dhalf], w)
        return 0
    lax.fori_loop(0, D // 2, body_d0, 0)


def _k1_body(emb_t, tail, out, in_a, in_b, out_a, out_b, tail_v,
             sem_a, sem_b, sem_oa, sem_ob):
    wid = lax.axis_index("s") * _NC + lax.axis_index("c")
    nst = jnp.where(wid < _NST - (_NST // _NW) * _NW, _NST // _NW + 1,
                    _NST // _NW)
    iota = lax.iota(jnp.int32, 16)
    iota_d = iota * (D // 2)
    pltpu.async_copy(emb_t.at[:, pl.ds(wid * _TW, _TW)], in_a, sem_a)

    def pair(j, _):
        ia = 2 * j
        ib = 2 * j + 1
        sa = wid + _NW * ia
        sb = wid + _NW * ib

        @pl.when(ib < nst)
        def _():
            pltpu.async_copy(emb_t.at[:, pl.ds(sb * _TW, _TW)], in_b, sem_b)

        pltpu.make_async_copy(emb_t.at[:, pl.ds(0, _TW)], in_a, sem_a).wait()

        @pl.when(j > 0)
        def _():
            pltpu.make_async_copy(
                out_a, out.at[pl.ds(0, _OUTW)], sem_oa).wait()

        _shuffle(in_a, out_a, iota, iota_d)
        pltpu.async_copy(out_a, out.at[pl.ds(sa * _OUTW, _OUTW)], sem_oa)

        @pl.when(ia + 2 < nst)
        def _():
            pltpu.async_copy(
                emb_t.at[:, pl.ds((wid + _NW * (ia + 2)) * _TW, _TW)],
                in_a, sem_a)

        @pl.when(ib < nst)
        def _():
            pltpu.make_async_copy(
                emb_t.at[:, pl.ds(0, _TW)], in_b, sem_b).wait()

            @pl.when(j > 0)
            def _():
                pltpu.make_async_copy(
                    out_b, out.at[pl.ds(0, _OUTW)], sem_ob).wait()

            _shuffle(in_b, out_b, iota, iota_d)
            pltpu.async_copy(out_b, out.at[pl.ds(sb * _OUTW, _OUTW)], sem_ob)

        return 0

    lax.fori_loop(0, _NPAIR, pair, 0)
    pltpu.make_async_copy(out_a, out.at[pl.ds(0, _OUTW)], sem_oa).wait()
    pltpu.make_async_copy(out_b, out.at[pl.ds(0, _OUTW)], sem_ob).wait()

    @pl.when(wid == 0)
    def _():
        pltpu.sync_copy(tail, tail_v)
        pltpu.sync_copy(tail_v, out.at[pl.ds(_VFULL * D // 2, _TAIL * D // 2)])


_k1 = functools.partial(
    pl.kernel,
    out_type=jax.ShapeDtypeStruct((V * D // 2,), jnp.int32),
    mesh=plsc.VectorSubcoreMesh(core_axis_name="c", subcore_axis_name="s"),
    scratch_types=[
        pltpu.VMEM((D, _TW), jnp.float32),
        pltpu.VMEM((D, _TW), jnp.float32),
        pltpu.VMEM((_OUTW,), jnp.int32),
        pltpu.VMEM((_OUTW,), jnp.int32),
        pltpu.VMEM((_TAIL * D // 2,), jnp.int32),
        pltpu.SemaphoreType.DMA,
        pltpu.SemaphoreType.DMA,
        pltpu.SemaphoreType.DMA,
        pltpu.SemaphoreType.DMA,
    ],
    compiler_params=pltpu.CompilerParams(use_tc_tiling_on_sc=True,
                                         needs_layout_passes=False),
)(_k1_body)

_BPW = BF // _NW
_CH = 416
_NCHUNK = _BPW // _CH


def _sc_gather_body(idx_hbm, emb_tab, lin_tab, emb_out, lin_out,
                    idx_v, rows0, rows1, lin_v, sem_e0, sem_e1, sem_l):
    wid = lax.axis_index("s") * _NC + lax.axis_index("c")
    base = wid * _BPW
    pltpu.sync_copy(idx_hbm.at[pl.ds(base, _BPW)], idx_v)
    cp_l = pltpu.async_copy(lin_tab.at[idx_v], lin_v, sem_l)
    bufs = (rows0, rows1)
    sems = (sem_e0, sem_e1)
    cps = [None, None]
    cps[0] = pltpu.async_copy(
        emb_tab.at[idx_v.at[pl.ds(0, _CH)]], rows0, sem_e0)
    for j in range(_NCHUNK):
        cur = j % 2
        if j + 1 < _NCHUNK:
            nxt = (j + 1) % 2
            cps[nxt] = pltpu.async_copy(
                emb_tab.at[idx_v.at[pl.ds((j + 1) * _CH, _CH)]],
                bufs[nxt], sems[nxt])
        cps[cur].wait()
        pltpu.sync_copy(bufs[cur], emb_out.at[pl.ds(base + j * _CH, _CH)])
    cp_l.wait()
    pltpu.sync_copy(lin_v, lin_out.at[pl.ds(base, _BPW)])


_sc_gather = functools.partial(
    pl.kernel,
    out_type=[
        jax.ShapeDtypeStruct((BF, D // 2), jnp.int32),
        jax.ShapeDtypeStruct((BF,), jnp.float32),
    ],
    mesh=plsc.VectorSubcoreMesh(core_axis_name="c", subcore_axis_name="s"),
    scratch_types=[
        pltpu.VMEM((_BPW,), jnp.int32),
        pltpu.VMEM((_CH, D // 2), jnp.int32),
        pltpu.VMEM((_CH, D // 2), jnp.int32),
        pltpu.VMEM((_BPW,), jnp.float32),
        pltpu.SemaphoreType.DMA,
        pltpu.SemaphoreType.DMA,
        pltpu.SemaphoreType.DMA,
    ],
    compiler_params=pltpu.CompilerParams(use_tc_tiling_on_sc=False),
)(_sc_gather_body)


def _mlp_body(emb_ref, lin_ref, bias_ref, w1_ref, b1_ref, g1_ref, be1_ref,
              w2_ref, b2_ref, g2_ref, be2_ref, w3_ref, b3_ref, out_ref):
    eps = 1e-5
    h = emb_ref[...]
    w1 = w1_ref[...].astype(jnp.bfloat16)
    h = jnp.dot(h, w1, preferred_element_type=jnp.float32) + b1_ref[...]
    mu = jnp.mean(h, axis=0, keepdims=True)
    var = jnp.mean((h - mu) ** 2, axis=0, keepdims=True)
    h = g1_ref[...] * (h - mu) * lax.rsqrt(var + eps) + be1_ref[...]
    h = jnp.maximum(h, 0.0)
    h = jnp.dot(h, w2_ref[...], preferred_element_type=jnp.float32) + b2_ref[...]
    mu = jnp.mean(h, axis=0, keepdims=True)
    var = jnp.mean((h - mu) ** 2, axis=0, keepdims=True)
    h = g2_ref[...] * (h - mu) * lax.rsqrt(var + eps) + be2_ref[...]
    h = jnp.maximum(h, 0.0)
    deep = jnp.dot(h, w3_ref[...], preferred_element_type=jnp.float32) + b3_ref[...]
    wide = jnp.sum(lin_ref[...], axis=1, keepdims=True)
    out_ref[...] = jax.nn.sigmoid(bias_ref[...] + wide + deep)


_mlp = pl.pallas_call(
    _mlp_body,
    out_shape=jax.ShapeDtypeStruct((B, 1), jnp.float32),
)


def kernel(input, linear_table, bias, emb_table, W1, b1, g1, be1,
           W2, b2, g2, be2, W3, b3):
    idx = input.reshape(BF)
    tail_bf = emb_table[_VFULL:, :].astype(jnp.bfloat16)
    tail = lax.bitcast_convert_type(
        tail_bf.reshape(_TAIL * D // 2, 2), jnp.int32)
    flat = _k1(emb_table.T, tail)
    emb_flat, lin_flat = _sc_gather(idx, flat.reshape(V, D // 2),
                                    linear_table.reshape(V))
    h = lax.bitcast_convert_type(emb_flat, jnp.bfloat16).reshape(B, F * D)
    lin2 = lin_flat.reshape(B, F)
    return _mlp(h, lin2, bias.reshape(1, 1),
                W1, b1.reshape(1, D), g1.reshape(1, D), be1.reshape(1, D),
                W2, b2.reshape(1, D), g2.reshape(1, D), be2.reshape(1, D),
                W3, b3.reshape(1, 1))


# speedup vs baseline: 5.9106x; 5.9106x over previous
import functools

import jax
import jax.numpy as jnp
from jax import lax
from jax.experimental import pallas as pl
from jax.experimental.pallas import tpu as pltpu
from jax.experimental.pallas import tpu_sc as plsc

V = 1000000
F = 26
D = 32
B = 4096
BF = B * F

_NC = 2
_NS = 16
_NW = _NC * _NS

_TW = 512
_VFULL = (V // 128) * 128
_NST = _VFULL // _TW
_NPAIR = (_NST // _NW + 1 + 1) // 2
_TAIL = V - _VFULL
_OUTW = _TW * D


def _shuffle(in_buf, out_buf, iota, iota_d):
    def body_d0(d0, _):
        dvec = lax.rem(d0 + iota, D)
        for k in range(_TW // 16):
            v = plsc.load_gather(in_buf, [dvec, iota + k * 16])
            plsc.store_scatter(out_buf, [iota_d + (k * 16 * D) + dvec], v)
        return 0
    lax.fori_loop(0, D, body_d0, 0)


def _k1_body(emb_t, tail, out, in_a, in_b, out_a, out_b, tail_v,
             sem_a, sem_b, sem_oa, sem_ob):
    wid = lax.axis_index("s") * _NC + lax.axis_index("c")
    nst = jnp.where(wid < _NST - (_NST // _NW) * _NW, _NST // _NW + 1,
                    _NST // _NW)
    iota = lax.iota(jnp.int32, 16)
    iota_d = iota * D
    pltpu.async_copy(emb_t.at[:, pl.ds(wid * _TW, _TW)], in_a, sem_a)

    def pair(j, _):
        ia = 2 * j
        ib = 2 * j + 1
        sa = wid + _NW * ia
        sb = wid + _NW * ib

        @pl.when(ib < nst)
        def _():
            pltpu.async_copy(emb_t.at[:, pl.ds(sb * _TW, _TW)], in_b, sem_b)

        pltpu.make_async_copy(emb_t.at[:, pl.ds(0, _TW)], in_a, sem_a).wait()

        @pl.when(j > 0)
        def _():
            pltpu.make_async_copy(
                out_a, out.at[pl.ds(0, _OUTW)], sem_oa).wait()

        _shuffle(in_a, out_a, iota, iota_d)
        pltpu.async_copy(out_a, out.at[pl.ds(sa * _OUTW, _OUTW)], sem_oa)

        @pl.when(ia + 2 < nst)
        def _():
            pltpu.async_copy(
                emb_t.at[:, pl.ds((wid + _NW * (ia + 2)) * _TW, _TW)],
                in_a, sem_a)

        @pl.when(ib < nst)
        def _():
            pltpu.make_async_copy(
                emb_t.at[:, pl.ds(0, _TW)], in_b, sem_b).wait()

            @pl.when(j > 0)
            def _():
                pltpu.make_async_copy(
                    out_b, out.at[pl.ds(0, _OUTW)], sem_ob).wait()

            _shuffle(in_b, out_b, iota, iota_d)
            pltpu.async_copy(out_b, out.at[pl.ds(sb * _OUTW, _OUTW)], sem_ob)

        return 0

    lax.fori_loop(0, _NPAIR, pair, 0)
    pltpu.make_async_copy(out_a, out.at[pl.ds(0, _OUTW)], sem_oa).wait()
    pltpu.make_async_copy(out_b, out.at[pl.ds(0, _OUTW)], sem_ob).wait()

    @pl.when(wid == 0)
    def _():
        pltpu.sync_copy(tail, tail_v)
        pltpu.sync_copy(tail_v, out.at[pl.ds(_VFULL * D, _TAIL * D)])


_k1 = functools.partial(
    pl.kernel,
    out_type=jax.ShapeDtypeStruct((V * D,), jnp.float32),
    mesh=plsc.VectorSubcoreMesh(core_axis_name="c", subcore_axis_name="s"),
    scratch_types=[
        pltpu.VMEM((D, _TW), jnp.float32),
        pltpu.VMEM((D, _TW), jnp.float32),
        pltpu.VMEM((_OUTW,), jnp.float32),
        pltpu.VMEM((_OUTW,), jnp.float32),
        pltpu.VMEM((_TAIL * D,), jnp.float32),
        pltpu.SemaphoreType.DMA,
        pltpu.SemaphoreType.DMA,
        pltpu.SemaphoreType.DMA,
        pltpu.SemaphoreType.DMA,
    ],
    compiler_params=pltpu.CompilerParams(use_tc_tiling_on_sc=True,
                                         needs_layout_passes=False),
)(_k1_body)

_BPW = BF // _NW
_CH = 416
_NCHUNK = _BPW // _CH


def _sc_gather_body(idx_hbm, emb_tab, lin_tab, emb_out, lin_out,
                    idx_v, rows0, rows1, lin_v, sem_e0, sem_e1, sem_l):
    wid = lax.axis_index("s") * _NC + lax.axis_index("c")
    base = wid * _BPW
    pltpu.sync_copy(idx_hbm.at[pl.ds(base, _BPW)], idx_v)
    cp_l = pltpu.async_copy(lin_tab.at[idx_v], lin_v, sem_l)
    bufs = (rows0, rows1)
    sems = (sem_e0, sem_e1)
    cps = [None, None]
    cps[0] = pltpu.async_copy(
        emb_tab.at[idx_v.at[pl.ds(0, _CH)]], rows0, sem_e0)
    for j in range(_NCHUNK):
        cur = j % 2
        if j + 1 < _NCHUNK:
            nxt = (j + 1) % 2
            cps[nxt] = pltpu.async_copy(
                emb_tab.at[idx_v.at[pl.ds((j + 1) * _CH, _CH)]],
                bufs[nxt], sems[nxt])
        cps[cur].wait()
        pltpu.sync_copy(bufs[cur], emb_out.at[pl.ds(base + j * _CH, _CH)])
    cp_l.wait()
    pltpu.sync_copy(lin_v, lin_out.at[pl.ds(base, _BPW)])


_sc_gather = functools.partial(
    pl.kernel,
    out_type=[
        jax.ShapeDtypeStruct((BF, D), jnp.float32),
        jax.ShapeDtypeStruct((BF,), jnp.float32),
    ],
    mesh=plsc.VectorSubcoreMesh(core_axis_name="c", subcore_axis_name="s"),
    scratch_types=[
        pltpu.VMEM((_BPW,), jnp.int32),
        pltpu.VMEM((_CH, D), jnp.float32),
        pltpu.VMEM((_CH, D), jnp.float32),
        pltpu.VMEM((_BPW,), jnp.float32),
        pltpu.SemaphoreType.DMA,
        pltpu.SemaphoreType.DMA,
        pltpu.SemaphoreType.DMA,
    ],
    compiler_params=pltpu.CompilerParams(use_tc_tiling_on_sc=False),
)(_sc_gather_body)


def _mlp_body(emb_ref, lin_ref, bias_ref, w1_ref, b1_ref, g1_ref, be1_ref,
              w2_ref, b2_ref, g2_ref, be2_ref, w3_ref, b3_ref, out_ref):
    eps = 1e-5
    h = emb_ref[...]
    h = jnp.dot(h, w1_ref[...], preferred_element_type=jnp.float32) + b1_ref[...]
    mu = jnp.mean(h, axis=0, keepdims=True)
    var = jnp.mean((h - mu) ** 2, axis=0, keepdims=True)
    h = g1_ref[...] * (h - mu) * lax.rsqrt(var + eps) + be1_ref[...]
    h = jnp.maximum(h, 0.0)
    h = jnp.dot(h, w2_ref[...], preferred_element_type=jnp.float32) + b2_ref[...]
    mu = jnp.mean(h, axis=0, keepdims=True)
    var = jnp.mean((h - mu) ** 2, axis=0, keepdims=True)
    h = g2_ref[...] * (h - mu) * lax.rsqrt(var + eps) + be2_ref[...]
    h = jnp.maximum(h, 0.0)
    deep = jnp.dot(h, w3_ref[...], preferred_element_type=jnp.float32) + b3_ref[...]
    wide = jnp.sum(lin_ref[...], axis=1, keepdims=True)
    out_ref[...] = jax.nn.sigmoid(bias_ref[...] + wide + deep)


_mlp = pl.pallas_call(
    _mlp_body,
    out_shape=jax.ShapeDtypeStruct((B, 1), jnp.float32),
)


def kernel(input, linear_table, bias, emb_table, W1, b1, g1, be1,
           W2, b2, g2, be2, W3, b3):
    idx = input.reshape(BF)
    tail = emb_table[_VFULL:, :].reshape(_TAIL * D)
    flat = _k1(emb_table.T, tail)
    emb_flat, lin_flat = _sc_gather(idx, flat.reshape(V, D),
                                    linear_table.reshape(V))
    h = emb_flat.reshape(B, F * D)
    lin2 = lin_flat.reshape(B, F)
    return _mlp(h, lin2, bias.reshape(1, 1),
                W1, b1.reshape(1, D), g1.reshape(1, D), be1.reshape(1, D),
                W2, b2.reshape(1, D), g2.reshape(1, D), be2.reshape(1, D),
                W3, b3.reshape(1, 1))
